# baseline (speedup 1.0000x reference)
import functools

import jax
import jax.numpy as jnp
from jax import lax
from jax.experimental import pallas as pl
from jax.experimental.pallas import tpu as pltpu
from jax.experimental.pallas import tpu_sc as plsc

N = 10000
E = 320000
D_IN = 128
H = 256
L = 5
G = 64
BN_EPS = 1e-5

HC = H // 2
NSC = 2
NT = 16
CH = 64
NCH = 315
STG = 63
NSTG = NCH // STG
EPP = NT * NCH * CH
NP = 10240
RPT = NP // NT
WB = 64
NWB = RPT // WB

BN_NODES = 1000
GRID = N // BN_NODES


def _sc_segment_sum(h2, srcp, dst4, zrows):
    mesh = plsc.VectorSubcoreMesh(
        core_axis_name="c", subcore_axis_name="s", num_cores=NSC,
        num_subcores=NT)

    @functools.partial(
        pl.kernel,
        out_type=jax.ShapeDtypeStruct((2 * NP, HC), jnp.float32),
        mesh=mesh,
        scratch_types=[
            pltpu.VMEM_SHARED((NP, HC), jnp.float32),
            pltpu.VMEM((STG, CH), jnp.int32),
            pltpu.VMEM((STG, CH), jnp.int32),
            pltpu.VMEM((CH, HC), jnp.float32),
            pltpu.VMEM((CH, HC), jnp.float32),
            pltpu.VMEM((CH, HC), jnp.float32),
            pltpu.VMEM((CH, HC), jnp.float32),
            pltpu.SemaphoreType.DMA,
            pltpu.SemaphoreType.DMA,
            pltpu.SemaphoreType.DMA,
            pltpu.SemaphoreType.DMA,
            pltpu.SemaphoreType.DMA,
            pltpu.SemaphoreType.DMA,
            pltpu.SemaphoreType.DMA,
            pltpu.SemaphoreType.DMA,
        ],
    )
    def k(h_hbm, src_hbm, dst_hbm, zr_hbm, out_hbm,
          acc, srcb, dstb, rows0, rows1, rows2, rows3,
          g0, g1, g2, g3, s0, s1, s2, s3):
        c = lax.axis_index("c")
        s = lax.axis_index("s")
        rows = (rows0, rows1, rows2, rows3)
        gsem = (g0, g1, g2, g3)
        ssem = (s0, s1, s2, s3)

        pltpu.sync_copy(zr_hbm, rows0.at[pl.ds(0, WB)])
        for r in range(NWB):
            pltpu.sync_copy(rows0.at[pl.ds(0, WB)],
                            acc.at[pl.ds(s * RPT + r * WB, WB)])
        plsc.subcore_barrier()

        def outer(kk, carry):
            pltpu.sync_copy(src_hbm.at[c, s, kk], srcb)
            pltpu.sync_copy(dst_hbm.at[s, kk], dstb)

            gd = [None, None, None, None]
            sd = [None, None, None, None]
            gd[0] = pltpu.async_copy(h_hbm.at[srcb.at[0]], rows[0], gsem[0])
            gd[1] = pltpu.async_copy(h_hbm.at[srcb.at[1]], rows[1], gsem[1])
            for j in range(STG):
                b = j % 4
                gd[b].wait()
                sd[b] = pltpu.async_copy(
                    rows[b], acc.at[dstb.at[j]], ssem[b], add=True)
                jn = j + 2
                if jn < STG:
                    bn = jn % 4
                    if j - 2 >= 0:
                        sd[(j - 2) % 4].wait()
                    gd[bn] = pltpu.async_copy(
                        h_hbm.at[srcb.at[jn]], rows[bn], gsem[bn])
            for t in range(max(0, STG - 4), STG):
                sd[t % 4].wait()
            return carry

        lax.fori_loop(0, NSTG, outer, 0)
        plsc.subcore_barrier()

        for r in range(NWB):
            off = s * RPT + r * WB
            pltpu.sync_copy(acc.at[pl.ds(off, WB)], rows0.at[pl.ds(0, WB)])
            pltpu.sync_copy(rows0.at[pl.ds(0, WB)],
                            out_hbm.at[pl.ds(c * NP + off, WB)])

    return k(h2, srcp, dst4, zrows)


def _tc_embed(x, w, brow):
    def body(x_ref, w_ref, b_ref, o_ref):
        h = jnp.dot(x_ref[...], w_ref[...],
                    preferred_element_type=jnp.float32) + b_ref[0:1, :]
        o_ref[0] = h[:, :HC]
        o_ref[1] = h[:, HC:]

    return pl.pallas_call(
        body,
        grid=(GRID,),
        in_specs=[
            pl.BlockSpec((BN_NODES, D_IN), lambda i: (i, 0)),
            pl.BlockSpec((D_IN, H), lambda i: (0, 0)),
            pl.BlockSpec((8, H), lambda i: (0, 0)),
        ],
        out_specs=pl.BlockSpec((2, BN_NODES, HC), lambda i: (0, i, 0)),
        out_shape=jax.ShapeDtypeStruct((2, N, HC), jnp.float32),
    )(x, w, brow)


def _tc_mlp(h, agg, w1, w2, pmat, skip):
    has_skip = skip is not None

    def body(*refs):
        if has_skip:
            h_ref, a_ref, w1_ref, w2_ref, p_ref, s_ref, o_ref = refs
        else:
            h_ref, a_ref, w1_ref, w2_ref, p_ref, o_ref = refs
        hb = jnp.concatenate([h_ref[0], h_ref[1]], axis=1)
        ab = jnp.concatenate([a_ref[0], a_ref[1]], axis=1)
        z = hb * p_ref[4:5, :] + ab
        t = jnp.dot(z, w1_ref[...], preferred_element_type=jnp.float32)
        t = jnp.maximum(t + p_ref[0:1, :], 0.0)
        t = jnp.dot(t, w2_ref[...], preferred_element_type=jnp.float32)
        t = jnp.maximum(t + p_ref[1:2, :], 0.0)
        hn = t + hb
        hn = jnp.maximum(hn * p_ref[2:3, :] + p_ref[3:4, :], 0.0)
        if has_skip:
            hn = hn + jnp.concatenate([s_ref[0], s_ref[1]], axis=1)
        o_ref[0] = hn[:, :HC]
        o_ref[1] = hn[:, HC:]

    in_specs = [
        pl.BlockSpec((2, BN_NODES, HC), lambda i: (0, i, 0)),
        pl.BlockSpec((2, BN_NODES, HC), lambda i: (0, i, 0)),
        pl.BlockSpec((H, H), lambda i: (0, 0)),
        pl.BlockSpec((H, H), lambda i: (0, 0)),
        pl.BlockSpec((8, H), lambda i: (0, 0)),
    ]
    args = [h, agg, w1, w2, pmat]
    if has_skip:
        in_specs.append(pl.BlockSpec((2, BN_NODES, HC), lambda i: (0, i, 0)))
        args.append(skip)

    return pl.pallas_call(
        body,
        grid=(GRID,),
        in_specs=in_specs,
        out_specs=pl.BlockSpec((2, BN_NODES, HC), lambda i: (0, i, 0)),
        out_shape=jax.ShapeDtypeStruct((2, N, HC), jnp.float32),
    )(*args)


def _tc_pool(h, bidx3):
    def body(h_ref, b_ref, o_ref, acc_ref):
        i = pl.program_id(0)

        @pl.when(i == 0)
        def _():
            acc_ref[...] = jnp.zeros_like(acc_ref)

        hb = jnp.concatenate(
            [h_ref[0], h_ref[1],
             jnp.ones((BN_NODES, HC), jnp.float32)], axis=1)
        onehot_t = (b_ref[0] == lax.broadcasted_iota(
            jnp.int32, (G, BN_NODES), 0)).astype(jnp.float32)
        acc_ref[...] += lax.dot_general(
            onehot_t, hb, (((1,), (0,)), ((), ())),
            preferred_element_type=jnp.float32)

        @pl.when(i == GRID - 1)
        def _():
            cnt = jnp.maximum(acc_ref[:, H:H + 1], 1.0)
            o_ref[...] = acc_ref[:, :H] / cnt

    return pl.pallas_call(
        body,
        grid=(GRID,),
        in_specs=[
            pl.BlockSpec((2, BN_NODES, HC), lambda i: (0, i, 0)),
            pl.BlockSpec((1, 1, BN_NODES), lambda i: (i, 0, 0)),
        ],
        out_specs=pl.BlockSpec((G, H), lambda i: (0, 0)),
        out_shape=jax.ShapeDtypeStruct((G, H), jnp.float32),
        scratch_shapes=[pltpu.VMEM((G, H + HC), jnp.float32)],
    )(h, bidx3)


def kernel(x, edge_index, batch_idx, params):
    inv = 1.0 / jnp.sqrt(jnp.float32(1.0 + BN_EPS))

    src = edge_index[0]
    dst = edge_index[1]
    pad = EPP - E
    src_p = jnp.concatenate([src, jnp.zeros((pad,), jnp.int32)])
    dst_p = jnp.concatenate(
        [dst, N + (jnp.arange(pad, dtype=jnp.int32) % (NP - N))])
    src4 = src_p.reshape(NT, NSTG, STG, CH)
    srcp = jnp.stack([src4, src4 + N])
    dst4 = dst_p.reshape(NT, NSTG, STG, CH)
    zrows = jnp.zeros((WB, HC), jnp.float32)
    bidx3 = batch_idx.reshape(GRID, 1, BN_NODES)

    brow = jnp.zeros((8, H), jnp.float32).at[0].set(params["emb_b"])
    h = _tc_embed(x, params["emb_W"], brow)

    outs = [h]
    for i in range(L):
        p = params["layers"][i]
        s1 = p["g1"] * inv
        s2 = p["g2"] * inv
        w1 = p["W1"] * s1[None, :]
        w2 = p["W2"] * s2[None, :]
        c1 = p["b1"] * s1 + p["be1"]
        c2 = p["b2"] * s2 + p["be2"]
        ob = params["outer_bn"][i]
        so = ob["g"] * inv
        bo = ob["b"]
        epsv = jnp.full((H,), 1.0, jnp.float32) * (1.0 + p["eps"])
        pmat = jnp.concatenate(
            [jnp.stack([c1, c2, so, bo, epsv]),
             jnp.zeros((3, H), jnp.float32)], axis=0)

        agg = _sc_segment_sum(h.reshape(2 * N, HC), srcp, dst4, zrows)
        skip = outs[-2] if (i > 0 and i % 2 == 1) else None
        h = _tc_mlp(h, agg.reshape(2, NP, HC), w1, w2, pmat, skip)
        outs.append(h)

    return _tc_pool(h, bidx3)


# speedup vs baseline: 1.4320x; 1.4320x over previous
import functools

import jax
import jax.numpy as jnp
from jax import lax
from jax.experimental import pallas as pl
from jax.experimental.pallas import tpu as pltpu
from jax.experimental.pallas import tpu_sc as plsc

N = 10000
E = 320000
D_IN = 128
H = 256
L = 5
G = 64
BN_EPS = 1e-5

HC = H // 2
NSC = 2
NT = 16
CH = 80
NCH = 250
STG = 25
NSTG = NCH // STG
EPP = NT * NCH * CH
NP = 10240
RPT = NP // NT
WB = 64
NWB = RPT // WB

BN_NODES = 1000
GRID = N // BN_NODES


def _sc_segment_sum(h2, srcp, dst4, zrows):
    mesh = plsc.VectorSubcoreMesh(
        core_axis_name="c", subcore_axis_name="s", num_cores=NSC,
        num_subcores=NT)

    @functools.partial(
        pl.kernel,
        out_type=jax.ShapeDtypeStruct((2 * NP, HC), jnp.float32),
        mesh=mesh,
        scratch_types=[
            pltpu.VMEM_SHARED((NP, HC), jnp.float32),
            pltpu.VMEM((STG, CH), jnp.int32),
            pltpu.VMEM((STG, CH), jnp.int32),
            pltpu.VMEM((CH, HC), jnp.float32),
            pltpu.VMEM((CH, HC), jnp.float32),
            pltpu.VMEM((CH, HC), jnp.float32),
            pltpu.VMEM((CH, HC), jnp.float32),
            pltpu.SemaphoreType.DMA,
            pltpu.SemaphoreType.DMA,
            pltpu.SemaphoreType.DMA,
            pltpu.SemaphoreType.DMA,
            pltpu.SemaphoreType.DMA,
            pltpu.SemaphoreType.DMA,
            pltpu.SemaphoreType.DMA,
            pltpu.SemaphoreType.DMA,
        ],
    )
    def k(h_hbm, src_hbm, dst_hbm, zr_hbm, out_hbm,
          acc, srcb, dstb, rows0, rows1, rows2, rows3,
          g0, g1, g2, g3, s0, s1, s2, s3):
        c = lax.axis_index("c")
        s = lax.axis_index("s")
        rows = (rows0, rows1, rows2, rows3)
        gsem = (g0, g1, g2, g3)
        ssem = (s0, s1, s2, s3)

        pltpu.sync_copy(zr_hbm, acc.at[pl.ds(s * RPT, RPT)])
        plsc.subcore_barrier()

        def outer(kk, carry):
            pltpu.sync_copy(src_hbm.at[c, s, kk], srcb)
            pltpu.sync_copy(dst_hbm.at[s, kk], dstb)

            gd = [None, None, None, None]
            sd = [None, None, None, None]
            gd[0] = pltpu.async_copy(h_hbm.at[srcb.at[0]], rows[0], gsem[0])
            gd[1] = pltpu.async_copy(h_hbm.at[srcb.at[1]], rows[1], gsem[1])
            for j in range(STG):
                b = j % 4
                gd[b].wait()
                sd[b] = pltpu.async_copy(
                    rows[b], acc.at[dstb.at[j]], ssem[b], add=True)
                jn = j + 2
                if jn < STG:
                    bn = jn % 4
                    if j - 2 >= 0:
                        sd[(j - 2) % 4].wait()
                    gd[bn] = pltpu.async_copy(
                        h_hbm.at[srcb.at[jn]], rows[bn], gsem[bn])
            for t in range(max(0, STG - 4), STG):
                sd[t % 4].wait()
            return carry

        lax.fori_loop(0, NSTG, outer, 0)
        plsc.subcore_barrier()

        pltpu.sync_copy(acc.at[pl.ds(s * RPT, RPT)],
                        out_hbm.at[pl.ds(c * NP + s * RPT, RPT)])

    return k(h2, srcp, dst4, zrows)


def _tc_embed(x, w, brow):
    def body(x_ref, w_ref, b_ref, o_ref):
        h = jnp.dot(x_ref[...], w_ref[...],
                    preferred_element_type=jnp.float32) + b_ref[0:1, :]
        o_ref[0] = h[:, :HC]
        o_ref[1] = h[:, HC:]

    return pl.pallas_call(
        body,
        grid=(GRID,),
        in_specs=[
            pl.BlockSpec((BN_NODES, D_IN), lambda i: (i, 0)),
            pl.BlockSpec((D_IN, H), lambda i: (0, 0)),
            pl.BlockSpec((8, H), lambda i: (0, 0)),
        ],
        out_specs=pl.BlockSpec((2, BN_NODES, HC), lambda i: (0, i, 0)),
        out_shape=jax.ShapeDtypeStruct((2, N, HC), jnp.float32),
    )(x, w, brow)


def _tc_mlp(h, agg, w1, w2, pmat, skip):
    has_skip = skip is not None

    def body(*refs):
        if has_skip:
            h_ref, a_ref, w1_ref, w2_ref, p_ref, s_ref, o_ref = refs
        else:
            h_ref, a_ref, w1_ref, w2_ref, p_ref, o_ref = refs
        hb = jnp.concatenate([h_ref[0], h_ref[1]], axis=1)
        ab = jnp.concatenate([a_ref[0], a_ref[1]], axis=1)
        z = hb * p_ref[4:5, :] + ab
        t = jnp.dot(z, w1_ref[...], preferred_element_type=jnp.float32)
        t = jnp.maximum(t + p_ref[0:1, :], 0.0)
        t = jnp.dot(t, w2_ref[...], preferred_element_type=jnp.float32)
        t = jnp.maximum(t + p_ref[1:2, :], 0.0)
        hn = t + hb
        hn = jnp.maximum(hn * p_ref[2:3, :] + p_ref[3:4, :], 0.0)
        if has_skip:
            hn = hn + jnp.concatenate([s_ref[0], s_ref[1]], axis=1)
        o_ref[0] = hn[:, :HC]
        o_ref[1] = hn[:, HC:]

    in_specs = [
        pl.BlockSpec((2, BN_NODES, HC), lambda i: (0, i, 0)),
        pl.BlockSpec((2, BN_NODES, HC), lambda i: (0, i, 0)),
        pl.BlockSpec((H, H), lambda i: (0, 0)),
        pl.BlockSpec((H, H), lambda i: (0, 0)),
        pl.BlockSpec((8, H), lambda i: (0, 0)),
    ]
    args = [h, agg, w1, w2, pmat]
    if has_skip:
        in_specs.append(pl.BlockSpec((2, BN_NODES, HC), lambda i: (0, i, 0)))
        args.append(skip)

    return pl.pallas_call(
        body,
        grid=(GRID,),
        in_specs=in_specs,
        out_specs=pl.BlockSpec((2, BN_NODES, HC), lambda i: (0, i, 0)),
        out_shape=jax.ShapeDtypeStruct((2, N, HC), jnp.float32),
    )(*args)


def _tc_pool(h, bidx3):
    def body(h_ref, b_ref, o_ref, acc_ref):
        i = pl.program_id(0)

        @pl.when(i == 0)
        def _():
            acc_ref[...] = jnp.zeros_like(acc_ref)

        hb = jnp.concatenate(
            [h_ref[0], h_ref[1],
             jnp.ones((BN_NODES, HC), jnp.float32)], axis=1)
        onehot_t = (b_ref[0] == lax.broadcasted_iota(
            jnp.int32, (G, BN_NODES), 0)).astype(jnp.float32)
        acc_ref[...] += lax.dot_general(
            onehot_t, hb, (((1,), (0,)), ((), ())),
            preferred_element_type=jnp.float32)

        @pl.when(i == GRID - 1)
        def _():
            cnt = jnp.maximum(acc_ref[:, H:H + 1], 1.0)
            o_ref[...] = acc_ref[:, :H] / cnt

    return pl.pallas_call(
        body,
        grid=(GRID,),
        in_specs=[
            pl.BlockSpec((2, BN_NODES, HC), lambda i: (0, i, 0)),
            pl.BlockSpec((1, 1, BN_NODES), lambda i: (i, 0, 0)),
        ],
        out_specs=pl.BlockSpec((G, H), lambda i: (0, 0)),
        out_shape=jax.ShapeDtypeStruct((G, H), jnp.float32),
        scratch_shapes=[pltpu.VMEM((G, H + HC), jnp.float32)],
    )(h, bidx3)


def kernel(x, edge_index, batch_idx, params):
    inv = 1.0 / jnp.sqrt(jnp.float32(1.0 + BN_EPS))

    src = edge_index[0]
    dst = edge_index[1]
    pad = EPP - E
    src_p = jnp.concatenate([src, jnp.zeros((pad,), jnp.int32)])
    dst_p = jnp.concatenate(
        [dst, N + (jnp.arange(pad, dtype=jnp.int32) % (NP - N))])
    src4 = src_p.reshape(NT, NSTG, STG, CH)
    srcp = jnp.stack([src4, src4 + N])
    dst4 = dst_p.reshape(NT, NSTG, STG, CH)
    zrows = jnp.zeros((RPT, HC), jnp.float32)
    bidx3 = batch_idx.reshape(GRID, 1, BN_NODES)

    brow = jnp.zeros((8, H), jnp.float32).at[0].set(params["emb_b"])
    h = _tc_embed(x, params["emb_W"], brow)

    outs = [h]
    for i in range(L):
        p = params["layers"][i]
        s1 = p["g1"] * inv
        s2 = p["g2"] * inv
        w1 = p["W1"] * s1[None, :]
        w2 = p["W2"] * s2[None, :]
        c1 = p["b1"] * s1 + p["be1"]
        c2 = p["b2"] * s2 + p["be2"]
        ob = params["outer_bn"][i]
        so = ob["g"] * inv
        bo = ob["b"]
        epsv = jnp.full((H,), 1.0, jnp.float32) * (1.0 + p["eps"])
        pmat = jnp.concatenate(
            [jnp.stack([c1, c2, so, bo, epsv]),
             jnp.zeros((3, H), jnp.float32)], axis=0)

        agg = _sc_segment_sum(h.reshape(2 * N, HC), srcp, dst4, zrows)
        skip = outs[-2] if (i > 0 and i % 2 == 1) else None
        h = _tc_mlp(h, agg.reshape(2, NP, HC), w1, w2, pmat, skip)
        outs.append(h)

    return _tc_pool(h, bidx3)


# speedup vs baseline: 1.4558x; 1.0166x over previous
import functools

import jax
import jax.numpy as jnp
from jax import lax
from jax.experimental import pallas as pl
from jax.experimental.pallas import tpu as pltpu
from jax.experimental.pallas import tpu_sc as plsc

N = 10000
E = 320000
D_IN = 128
H = 256
L = 5
G = 64
BN_EPS = 1e-5

HC = H // 2
NSC = 2
NT = 16
CH = 80
NCH = 250
STG = 25
NSTG = NCH // STG
EPP = NT * NCH * CH
NP = 10240
RPT = NP // NT
WB = 64
NWB = RPT // WB

BN_NODES = 1000
GRID = N // BN_NODES


def _sc_segment_sum(h2, srcp, dst4, zrows):
    mesh = plsc.VectorSubcoreMesh(
        core_axis_name="c", subcore_axis_name="s", num_cores=NSC,
        num_subcores=NT)

    @functools.partial(
        pl.kernel,
        out_type=jax.ShapeDtypeStruct((2 * NP, HC), jnp.float32),
        mesh=mesh,
        scratch_types=[
            pltpu.VMEM_SHARED((NP, HC), jnp.float32),
            pltpu.VMEM((STG, CH), jnp.int32),
            pltpu.VMEM((STG, CH), jnp.int32),
            pltpu.VMEM((CH, HC), jnp.float32),
            pltpu.VMEM((CH, HC), jnp.float32),
            pltpu.VMEM((CH, HC), jnp.float32),
            pltpu.VMEM((CH, HC), jnp.float32),
            pltpu.SemaphoreType.DMA,
            pltpu.SemaphoreType.DMA,
            pltpu.SemaphoreType.DMA,
            pltpu.SemaphoreType.DMA,
            pltpu.SemaphoreType.DMA,
            pltpu.SemaphoreType.DMA,
            pltpu.SemaphoreType.DMA,
            pltpu.SemaphoreType.DMA,
            pltpu.SemaphoreType.DMA,
        ],
    )
    def k(h_hbm, src_hbm, dst_hbm, zr_hbm, out_hbm,
          acc, srcb, dstb, rows0, rows1, rows2, rows3,
          g0, g1, g2, g3, s0, s1, s2, s3, rs):
        c = lax.axis_index("c")
        s = lax.axis_index("s")
        rows = (rows0, rows1, rows2, rows3)
        gsem = (g0, g1, g2, g3)
        ssem = (s0, s1, s2, s3)

        pltpu.async_copy(src_hbm.at[c, s, 0], srcb, rs)
        pltpu.sync_copy(zr_hbm, acc.at[pl.ds(s * RPT, RPT)])
        plsc.subcore_barrier()

        def outer(kk, carry):
            pltpu.make_async_copy(src_hbm.at[c, s, kk], srcb, rs).wait()
            pltpu.sync_copy(dst_hbm.at[s, kk], dstb)

            gd = [None, None, None, None]
            sd = [None, None, None, None]
            gd[0] = pltpu.async_copy(h_hbm.at[srcb.at[0]], rows[0], gsem[0])
            gd[1] = pltpu.async_copy(h_hbm.at[srcb.at[1]], rows[1], gsem[1])
            for j in range(STG):
                b = j % 4
                gd[b].wait()
                sd[b] = pltpu.async_copy(
                    rows[b], acc.at[dstb.at[j]], ssem[b], add=True)
                jn = j + 2
                if jn < STG:
                    bn = jn % 4
                    if j - 2 >= 0:
                        sd[(j - 2) % 4].wait()
                    gd[bn] = pltpu.async_copy(
                        h_hbm.at[srcb.at[jn]], rows[bn], gsem[bn])
            @pl.when(kk + 1 < NSTG)
            def _():
                pltpu.async_copy(src_hbm.at[c, s, kk + 1], srcb, rs)

            for t in range(max(0, STG - 4), STG):
                sd[t % 4].wait()
            return carry

        lax.fori_loop(0, NSTG, outer, 0)
        plsc.subcore_barrier()

        pltpu.sync_copy(acc.at[pl.ds(s * RPT, RPT)],
                        out_hbm.at[pl.ds(c * NP + s * RPT, RPT)])

    return k(h2, srcp, dst4, zrows)


def _tc_embed(x, w, brow):
    def body(x_ref, w_ref, b_ref, o_ref):
        h = jnp.dot(x_ref[...], w_ref[...],
                    preferred_element_type=jnp.float32) + b_ref[0:1, :]
        o_ref[0] = h[:, :HC]
        o_ref[1] = h[:, HC:]

    return pl.pallas_call(
        body,
        grid=(GRID,),
        in_specs=[
            pl.BlockSpec((BN_NODES, D_IN), lambda i: (i, 0)),
            pl.BlockSpec((D_IN, H), lambda i: (0, 0)),
            pl.BlockSpec((8, H), lambda i: (0, 0)),
        ],
        out_specs=pl.BlockSpec((2, BN_NODES, HC), lambda i: (0, i, 0)),
        out_shape=jax.ShapeDtypeStruct((2, N, HC), jnp.float32),
    )(x, w, brow)


def _tc_mlp(h, agg, w1, w2, pmat, skip):
    has_skip = skip is not None

    def body(*refs):
        if has_skip:
            h_ref, a_ref, w1_ref, w2_ref, p_ref, s_ref, o_ref = refs
        else:
            h_ref, a_ref, w1_ref, w2_ref, p_ref, o_ref = refs
        hb = jnp.concatenate([h_ref[0], h_ref[1]], axis=1)
        ab = jnp.concatenate([a_ref[0], a_ref[1]], axis=1)
        z = hb * p_ref[4:5, :] + ab
        t = jnp.dot(z, w1_ref[...], preferred_element_type=jnp.float32)
        t = jnp.maximum(t + p_ref[0:1, :], 0.0)
        t = jnp.dot(t, w2_ref[...], preferred_element_type=jnp.float32)
        t = jnp.maximum(t + p_ref[1:2, :], 0.0)
        hn = t + hb
        hn = jnp.maximum(hn * p_ref[2:3, :] + p_ref[3:4, :], 0.0)
        if has_skip:
            hn = hn + jnp.concatenate([s_ref[0], s_ref[1]], axis=1)
        o_ref[0] = hn[:, :HC]
        o_ref[1] = hn[:, HC:]

    in_specs = [
        pl.BlockSpec((2, BN_NODES, HC), lambda i: (0, i, 0)),
        pl.BlockSpec((2, BN_NODES, HC), lambda i: (0, i, 0)),
        pl.BlockSpec((H, H), lambda i: (0, 0)),
        pl.BlockSpec((H, H), lambda i: (0, 0)),
        pl.BlockSpec((8, H), lambda i: (0, 0)),
    ]
    args = [h, agg, w1, w2, pmat]
    if has_skip:
        in_specs.append(pl.BlockSpec((2, BN_NODES, HC), lambda i: (0, i, 0)))
        args.append(skip)

    return pl.pallas_call(
        body,
        grid=(GRID,),
        in_specs=in_specs,
        out_specs=pl.BlockSpec((2, BN_NODES, HC), lambda i: (0, i, 0)),
        out_shape=jax.ShapeDtypeStruct((2, N, HC), jnp.float32),
    )(*args)


def _tc_pool(h, bidx3):
    def body(h_ref, b_ref, o_ref, acc_ref):
        i = pl.program_id(0)

        @pl.when(i == 0)
        def _():
            acc_ref[...] = jnp.zeros_like(acc_ref)

        hb = jnp.concatenate(
            [h_ref[0], h_ref[1],
             jnp.ones((BN_NODES, HC), jnp.float32)], axis=1)
        onehot_t = (b_ref[0] == lax.broadcasted_iota(
            jnp.int32, (G, BN_NODES), 0)).astype(jnp.float32)
        acc_ref[...] += lax.dot_general(
            onehot_t, hb, (((1,), (0,)), ((), ())),
            preferred_element_type=jnp.float32)

        @pl.when(i == GRID - 1)
        def _():
            cnt = jnp.maximum(acc_ref[:, H:H + 1], 1.0)
            o_ref[...] = acc_ref[:, :H] / cnt

    return pl.pallas_call(
        body,
        grid=(GRID,),
        in_specs=[
            pl.BlockSpec((2, BN_NODES, HC), lambda i: (0, i, 0)),
            pl.BlockSpec((1, 1, BN_NODES), lambda i: (i, 0, 0)),
        ],
        out_specs=pl.BlockSpec((G, H), lambda i: (0, 0)),
        out_shape=jax.ShapeDtypeStruct((G, H), jnp.float32),
        scratch_shapes=[pltpu.VMEM((G, H + HC), jnp.float32)],
    )(h, bidx3)


def kernel(x, edge_index, batch_idx, params):
    inv = 1.0 / jnp.sqrt(jnp.float32(1.0 + BN_EPS))

    src = edge_index[0]
    dst = edge_index[1]
    pad = EPP - E
    src_p = jnp.concatenate([src, jnp.zeros((pad,), jnp.int32)])
    dst_p = jnp.concatenate(
        [dst, N + (jnp.arange(pad, dtype=jnp.int32) % (NP - N))])
    src4 = src_p.reshape(NT, NSTG, STG, CH)
    srcp = jnp.stack([src4, src4 + N])
    dst4 = dst_p.reshape(NT, NSTG, STG, CH)
    zrows = jnp.zeros((RPT, HC), jnp.float32)
    bidx3 = batch_idx.reshape(GRID, 1, BN_NODES)

    brow = jnp.zeros((8, H), jnp.float32).at[0].set(params["emb_b"])
    h = _tc_embed(x, params["emb_W"], brow)

    outs = [h]
    for i in range(L):
        p = params["layers"][i]
        s1 = p["g1"] * inv
        s2 = p["g2"] * inv
        w1 = p["W1"] * s1[None, :]
        w2 = p["W2"] * s2[None, :]
        c1 = p["b1"] * s1 + p["be1"]
        c2 = p["b2"] * s2 + p["be2"]
        ob = params["outer_bn"][i]
        so = ob["g"] * inv
        bo = ob["b"]
        epsv = jnp.full((H,), 1.0, jnp.float32) * (1.0 + p["eps"])
        pmat = jnp.concatenate(
            [jnp.stack([c1, c2, so, bo, epsv]),
             jnp.zeros((3, H), jnp.float32)], axis=0)

        agg = _sc_segment_sum(h.reshape(2 * N, HC), srcp, dst4, zrows)
        skip = outs[-2] if (i > 0 and i % 2 == 1) else None
        h = _tc_mlp(h, agg.reshape(2, NP, HC), w1, w2, pmat, skip)
        outs.append(h)

    return _tc_pool(h, bidx3)
